# Initial kernel scaffold; baseline (speedup 1.0000x reference)
import jax
import jax.numpy as jnp
from jax.experimental import pallas as pl


def kernel(x, Wr, W1, b1, W2, b2):
    raise NotImplementedError("write your pallas kernel here")


# speedup vs baseline: 1.1208x; 1.1208x over previous
import functools

import jax
import jax.numpy as jnp
from jax import lax
from jax.experimental import pallas as pl
from jax.experimental.pallas import tpu as pltpu
from jax.experimental.pallas import tpu_sc as plsc

D_MODEL = 768
D_EXPERT = 1536
N_EXPERTS = 64
N_TOKENS = 2048
CAP = 40
N_SLOTS = N_EXPERTS * CAP
EO_ROWS = N_SLOTS + CAP
TB = 256
N_TB = N_TOKENS // TB


def _router_body(x_ref, wr_ref, top1_ref, psum_ref, z2_ref):
    i = pl.program_id(0)
    xb = x_ref[...]
    wr = wr_ref[...]
    logits = lax.dot_general(xb, wr, (((1,), (1,)), ((), ())),
                             preferred_element_type=jnp.float32)
    mx = jnp.max(logits, axis=1, keepdims=True)
    ex = jnp.exp(logits - mx)
    s = jnp.sum(ex, axis=1, keepdims=True)
    probs = ex / s
    logz = mx + jnp.log(s)
    z2 = jnp.sum(logz * logz)

    ids = lax.broadcasted_iota(jnp.int32, (TB, N_EXPERTS), 1)
    cand = jnp.where(logits == mx, ids, N_EXPERTS)
    top1 = jnp.min(cand, axis=1, keepdims=True)
    top1_ref[0] = top1

    psum = jnp.sum(probs, axis=0, keepdims=True).reshape(1, 1, N_EXPERTS)
    z2b = jnp.full((1, 1, 8), z2, dtype=jnp.float32)

    @pl.when(i == 0)
    def _():
        psum_ref[...] = psum
        z2_ref[...] = z2b

    @pl.when(i > 0)
    def _():
        psum_ref[...] += psum
        z2_ref[...] += z2b


def _router(x2d, wr):
    return pl.pallas_call(
        _router_body,
        grid=(N_TB,),
        in_specs=[
            pl.BlockSpec((TB, D_MODEL), lambda i: (i, 0)),
            pl.BlockSpec((N_EXPERTS, D_MODEL), lambda i: (0, 0)),
        ],
        out_specs=[
            pl.BlockSpec((1, TB, 1), lambda i: (i, 0, 0)),
            pl.BlockSpec((1, 1, N_EXPERTS), lambda i: (0, 0, 0)),
            pl.BlockSpec((1, 1, 8), lambda i: (0, 0, 0)),
        ],
        out_shape=[
            jax.ShapeDtypeStruct((N_TB, TB, 1), jnp.int32),
            jax.ShapeDtypeStruct((1, 1, N_EXPERTS), jnp.float32),
            jax.ShapeDtypeStruct((1, 1, 8), jnp.float32),
        ],
    )(x2d, wr)


def _assign_body(top1_hbm, slots_hbm, dest_hbm, kc_hbm,
                 tok_v, counts_v, slots_v, dest_v, kc_v):
    wid = lax.axis_index("s") * 2 + lax.axis_index("c")

    @pl.when(wid == 0)
    def _():
        pltpu.sync_copy(top1_hbm, tok_v)
        zeros16 = jnp.zeros((16,), jnp.int32)
        for j in range(N_EXPERTS // 16):
            counts_v[pl.ds(j * 16, 16)] = zeros16

        def zloop(j, _):
            slots_v[pl.ds(j * 16, 16)] = zeros16
            return 0
        lax.fori_loop(0, N_SLOTS // 16, zloop, 0)

        iota16 = lax.iota(jnp.int32, 16)

        def body(i, _):
            e = tok_v[pl.ds(i * 16, 16)]
            pre = plsc.load_gather(counts_v, [e])
            dup, lastm = plsc.scan_count(e)
            rank = pre + dup - 1
            plsc.store_scatter(counts_v, [e], rank + 1, mask=lastm)
            kept = rank < CAP
            d = e * CAP + rank
            n_ids = i * 16 + iota16
            plsc.store_scatter(slots_v, [jnp.where(kept, d, 0)], n_ids,
                               mask=kept)
            dest_v[pl.ds(i * 16, 16)] = jnp.where(kept, d, N_SLOTS)
            return 0
        lax.fori_loop(0, N_TOKENS // 16, body, 0)

        for j in range(N_EXPERTS // 16):
            c = counts_v[pl.ds(j * 16, 16)]
            kc_v[pl.ds(j * 16, 16)] = jnp.minimum(c, CAP)

        pltpu.sync_copy(slots_v, slots_hbm)
        pltpu.sync_copy(dest_v, dest_hbm)
        pltpu.sync_copy(kc_v, kc_hbm)


def _assign(top1):
    mesh = plsc.VectorSubcoreMesh(core_axis_name="c", subcore_axis_name="s")
    f = pl.kernel(
        _assign_body,
        out_type=(
            jax.ShapeDtypeStruct((N_SLOTS,), jnp.int32),
            jax.ShapeDtypeStruct((N_TOKENS,), jnp.int32),
            jax.ShapeDtypeStruct((N_EXPERTS,), jnp.int32),
        ),
        mesh=mesh,
        scratch_types=[
            pltpu.VMEM((N_TOKENS,), jnp.int32),
            pltpu.VMEM((N_EXPERTS,), jnp.int32),
            pltpu.VMEM((N_SLOTS,), jnp.int32),
            pltpu.VMEM((N_TOKENS,), jnp.int32),
            pltpu.VMEM((N_EXPERTS,), jnp.int32),
        ],
        compiler_params=pltpu.CompilerParams(needs_layout_passes=False),
    )
    return f(top1)


def _dispatch_body(rows_per, slots_hbm, x_hbm, xb_hbm, idx_v, rows_v, sem):
    wid = lax.axis_index("s") * 2 + lax.axis_index("c")
    base = wid * rows_per
    pltpu.sync_copy(slots_hbm.at[pl.ds(base, rows_per)], idx_v)
    pltpu.async_copy(x_hbm.at[idx_v], rows_v, sem).wait()
    pltpu.sync_copy(rows_v, xb_hbm.at[pl.ds(base, rows_per)])


def _dispatch(slots, x2d):
    info = plsc.get_sparse_core_info()
    nw = info.num_cores * info.num_subcores
    rows_per = N_SLOTS // nw
    mesh = plsc.VectorSubcoreMesh(core_axis_name="c", subcore_axis_name="s")
    f = pl.kernel(
        functools.partial(_dispatch_body, rows_per),
        out_type=jax.ShapeDtypeStruct((N_SLOTS, D_MODEL), jnp.float32),
        mesh=mesh,
        scratch_types=[
            pltpu.VMEM((rows_per,), jnp.int32),
            pltpu.VMEM((rows_per, D_MODEL), jnp.float32),
            pltpu.SemaphoreType.DMA,
        ],
    )
    return f(slots, x2d)


def _experts_body(xb_ref, w1_ref, b1_ref, w2_ref, b2_ref, eo_ref):
    e = pl.program_id(0)
    xb = xb_ref[...]
    w1 = w1_ref[0]
    h = lax.dot_general(xb, w1, (((1,), (1,)), ((), ())),
                        preferred_element_type=jnp.float32)
    h = jnp.maximum(h + b1_ref[0], 0.0)
    w2 = w2_ref[0]
    o = lax.dot_general(h, w2, (((1,), (1,)), ((), ())),
                        preferred_element_type=jnp.float32)
    o = o + b2_ref[0]
    eo_ref[...] = jnp.where(e < N_EXPERTS, o, 0.0)


def _experts(xb, w1, b1, w2, b2):
    b1r = b1.reshape(N_EXPERTS, 1, D_EXPERT)
    b2r = b2.reshape(N_EXPERTS, 1, D_MODEL)
    return pl.pallas_call(
        _experts_body,
        grid=(N_EXPERTS + 1,),
        in_specs=[
            pl.BlockSpec((CAP, D_MODEL),
                         lambda e: (jnp.minimum(e, N_EXPERTS - 1), 0)),
            pl.BlockSpec((1, D_EXPERT, D_MODEL),
                         lambda e: (jnp.minimum(e, N_EXPERTS - 1), 0, 0)),
            pl.BlockSpec((1, 1, D_EXPERT),
                         lambda e: (jnp.minimum(e, N_EXPERTS - 1), 0, 0)),
            pl.BlockSpec((1, D_MODEL, D_EXPERT),
                         lambda e: (jnp.minimum(e, N_EXPERTS - 1), 0, 0)),
            pl.BlockSpec((1, 1, D_MODEL),
                         lambda e: (jnp.minimum(e, N_EXPERTS - 1), 0, 0)),
        ],
        out_specs=pl.BlockSpec((CAP, D_MODEL), lambda e: (e, 0)),
        out_shape=jax.ShapeDtypeStruct((EO_ROWS, D_MODEL), jnp.float32),
        compiler_params=pltpu.CompilerParams(
            vmem_limit_bytes=100 * 1024 * 1024,
        ),
    )(xb, w1, b1r, w2, b2r)


def _combine_body(rows_per, dest_hbm, eo_hbm, out_hbm, idx_v, rows_v, sem):
    wid = lax.axis_index("s") * 2 + lax.axis_index("c")
    base = wid * rows_per
    pltpu.sync_copy(dest_hbm.at[pl.ds(base, rows_per)], idx_v)
    pltpu.async_copy(eo_hbm.at[idx_v], rows_v, sem).wait()
    pltpu.sync_copy(rows_v, out_hbm.at[pl.ds(base, rows_per)])


def _combine(dest, eo):
    info = plsc.get_sparse_core_info()
    nw = info.num_cores * info.num_subcores
    rows_per = N_TOKENS // nw
    mesh = plsc.VectorSubcoreMesh(core_axis_name="c", subcore_axis_name="s")
    f = pl.kernel(
        functools.partial(_combine_body, rows_per),
        out_type=jax.ShapeDtypeStruct((N_TOKENS, D_MODEL), jnp.float32),
        mesh=mesh,
        scratch_types=[
            pltpu.VMEM((rows_per,), jnp.int32),
            pltpu.VMEM((rows_per, D_MODEL), jnp.float32),
            pltpu.SemaphoreType.DMA,
        ],
    )
    return f(dest, eo)


def kernel(x, Wr, W1, b1, W2, b2):
    B, T, D = x.shape
    x2d = x.reshape(B * T, D)

    top1, psum, z2 = _router(x2d, Wr)
    top1 = top1.reshape(N_TOKENS)

    slots, dest, kc = _assign(top1)
    xb = _dispatch(slots, x2d)
    eo = _experts(xb, W1, b1, W2, b2)
    out2d = _combine(dest, eo)

    p_i = psum[0, 0, :] / jnp.float32(N_TOKENS)
    z_loss = z2[0, 0, 0] / jnp.float32(N_TOKENS)
    kcf = kc.astype(jnp.float32)
    total_sel = jnp.maximum(jnp.sum(kcf), 1.0)
    f_i = kcf / total_sel
    aux_loss = N_EXPERTS * jnp.sum(f_i * p_i)
    total_aux = 0.01 * aux_loss + 0.001 * z_loss

    out = out2d.reshape(B, T, D)
    return out, aux_loss, z_loss, total_aux


# speedup vs baseline: 1.2877x; 1.1488x over previous
import functools

import jax
import jax.numpy as jnp
from jax import lax
from jax.experimental import pallas as pl
from jax.experimental.pallas import tpu as pltpu
from jax.experimental.pallas import tpu_sc as plsc

D_MODEL = 768
D_EXPERT = 1536
N_EXPERTS = 64
N_TOKENS = 2048
CAP = 40
N_SLOTS = N_EXPERTS * CAP
EO_ROWS = N_SLOTS + CAP
TB = 256
N_TB = N_TOKENS // TB


def _router_body(x_ref, wr_ref, dest_ref, kc_ref, psum_ref, z2_ref, cnt_ref):
    i = pl.program_id(0)
    xb = x_ref[...]
    wr = wr_ref[...]
    logits = lax.dot_general(xb, wr, (((1,), (1,)), ((), ())),
                             preferred_element_type=jnp.float32)
    mx = jnp.max(logits, axis=1, keepdims=True)
    ex = jnp.exp(logits - mx)
    s = jnp.sum(ex, axis=1, keepdims=True)
    probs = ex / s
    logz = mx + jnp.log(s)
    z2 = jnp.sum(logz * logz)

    ids = lax.broadcasted_iota(jnp.int32, (TB, N_EXPERTS), 1)
    cand = jnp.where(logits == mx, ids, N_EXPERTS)
    top1 = jnp.min(cand, axis=1, keepdims=True)
    mask = (ids == top1).astype(jnp.float32)

    r_io = lax.broadcasted_iota(jnp.int32, (TB, TB), 0)
    c_io = lax.broadcasted_iota(jnp.int32, (TB, TB), 1)
    ltri = (c_io <= r_io).astype(jnp.float32)
    incl = lax.dot_general(ltri, mask, (((1,), (0,)), ((), ())),
                           preferred_element_type=jnp.float32)

    @pl.when(i == 0)
    def _():
        cnt_ref[...] = jnp.zeros((1, N_EXPERTS), jnp.float32)

    pos = cnt_ref[...] + incl - 1.0
    cnt_ref[...] += jnp.sum(mask, axis=0, keepdims=True)

    pos_tok = jnp.sum(mask * pos, axis=1)
    kept = pos_tok < CAP
    dest = jnp.where(kept,
                     top1[:, 0] * CAP + pos_tok.astype(jnp.int32),
                     N_SLOTS)
    dest_ref[...] = dest.reshape(1, 1, TB)

    psum = jnp.sum(probs, axis=0, keepdims=True).reshape(1, 1, N_EXPERTS)
    z2b = jnp.full((1, 1, 8), z2, dtype=jnp.float32)

    @pl.when(i == 0)
    def _():
        psum_ref[...] = psum
        z2_ref[...] = z2b

    @pl.when(i > 0)
    def _():
        psum_ref[...] += psum
        z2_ref[...] += z2b

    @pl.when(i == N_TB - 1)
    def _():
        kc_ref[...] = jnp.minimum(cnt_ref[...], float(CAP)).reshape(
            1, 1, N_EXPERTS)


def _router(x2d, wr):
    return pl.pallas_call(
        _router_body,
        grid=(N_TB,),
        in_specs=[
            pl.BlockSpec((TB, D_MODEL), lambda i: (i, 0)),
            pl.BlockSpec((N_EXPERTS, D_MODEL), lambda i: (0, 0)),
        ],
        out_specs=[
            pl.BlockSpec((1, 1, TB), lambda i: (i, 0, 0)),
            pl.BlockSpec((1, 1, N_EXPERTS), lambda i: (0, 0, 0)),
            pl.BlockSpec((1, 1, N_EXPERTS), lambda i: (0, 0, 0)),
            pl.BlockSpec((1, 1, 8), lambda i: (0, 0, 0)),
        ],
        out_shape=[
            jax.ShapeDtypeStruct((N_TB, 1, TB), jnp.int32),
            jax.ShapeDtypeStruct((1, 1, N_EXPERTS), jnp.float32),
            jax.ShapeDtypeStruct((1, 1, N_EXPERTS), jnp.float32),
            jax.ShapeDtypeStruct((1, 1, 8), jnp.float32),
        ],
        scratch_shapes=[pltpu.VMEM((1, N_EXPERTS), jnp.float32)],
    )(x2d, wr)


def _dispatch_body(rows_per, dest_hbm, x_hbm, xb_hbm, idx_v, rows_v, sem):
    wid = lax.axis_index("s") * 2 + lax.axis_index("c")
    base = wid * rows_per
    pltpu.sync_copy(dest_hbm.at[pl.ds(base, rows_per)], idx_v)
    pltpu.sync_copy(x_hbm.at[pl.ds(base, rows_per)], rows_v)
    pltpu.async_copy(rows_v, xb_hbm.at[idx_v], sem).wait()


def _dispatch(dest, x2d):
    info = plsc.get_sparse_core_info()
    nw = info.num_cores * info.num_subcores
    rows_per = N_TOKENS // nw
    mesh = plsc.VectorSubcoreMesh(core_axis_name="c", subcore_axis_name="s")
    f = pl.kernel(
        functools.partial(_dispatch_body, rows_per),
        out_type=jax.ShapeDtypeStruct((EO_ROWS, D_MODEL), jnp.float32),
        mesh=mesh,
        scratch_types=[
            pltpu.VMEM((rows_per,), jnp.int32),
            pltpu.VMEM((rows_per, D_MODEL), jnp.float32),
            pltpu.SemaphoreType.DMA,
        ],
    )
    return f(dest, x2d)


def _experts_body(xb_ref, w1_ref, b1_ref, w2_ref, b2_ref, eo_ref):
    e = pl.program_id(0)
    xb = xb_ref[...]
    w1 = w1_ref[0]
    h = lax.dot_general(xb, w1, (((1,), (1,)), ((), ())),
                        preferred_element_type=jnp.float32)
    h = jnp.maximum(h + b1_ref[0], 0.0)
    w2 = w2_ref[0]
    o = lax.dot_general(h, w2, (((1,), (1,)), ((), ())),
                        preferred_element_type=jnp.float32)
    o = o + b2_ref[0]
    eo_ref[...] = jnp.where(e < N_EXPERTS, o, 0.0)


def _experts(xb, w1, b1, w2, b2):
    b1r = b1.reshape(N_EXPERTS, 1, D_EXPERT)
    b2r = b2.reshape(N_EXPERTS, 1, D_MODEL)
    return pl.pallas_call(
        _experts_body,
        grid=(N_EXPERTS + 1,),
        in_specs=[
            pl.BlockSpec((CAP, D_MODEL),
                         lambda e: (jnp.minimum(e, N_EXPERTS - 1), 0)),
            pl.BlockSpec((1, D_EXPERT, D_MODEL),
                         lambda e: (jnp.minimum(e, N_EXPERTS - 1), 0, 0)),
            pl.BlockSpec((1, 1, D_EXPERT),
                         lambda e: (jnp.minimum(e, N_EXPERTS - 1), 0, 0)),
            pl.BlockSpec((1, D_MODEL, D_EXPERT),
                         lambda e: (jnp.minimum(e, N_EXPERTS - 1), 0, 0)),
            pl.BlockSpec((1, 1, D_MODEL),
                         lambda e: (jnp.minimum(e, N_EXPERTS - 1), 0, 0)),
        ],
        out_specs=pl.BlockSpec((CAP, D_MODEL), lambda e: (e, 0)),
        out_shape=jax.ShapeDtypeStruct((EO_ROWS, D_MODEL), jnp.float32),
        compiler_params=pltpu.CompilerParams(
            vmem_limit_bytes=100 * 1024 * 1024,
        ),
    )(xb, w1, b1r, w2, b2r)


def _combine_body(rows_per, dest_hbm, eo_hbm, out_hbm, idx_v, rows_v, sem):
    wid = lax.axis_index("s") * 2 + lax.axis_index("c")
    base = wid * rows_per
    pltpu.sync_copy(dest_hbm.at[pl.ds(base, rows_per)], idx_v)
    pltpu.async_copy(eo_hbm.at[idx_v], rows_v, sem).wait()
    pltpu.sync_copy(rows_v, out_hbm.at[pl.ds(base, rows_per)])


def _combine(dest, eo):
    info = plsc.get_sparse_core_info()
    nw = info.num_cores * info.num_subcores
    rows_per = N_TOKENS // nw
    mesh = plsc.VectorSubcoreMesh(core_axis_name="c", subcore_axis_name="s")
    f = pl.kernel(
        functools.partial(_combine_body, rows_per),
        out_type=jax.ShapeDtypeStruct((N_TOKENS, D_MODEL), jnp.float32),
        mesh=mesh,
        scratch_types=[
            pltpu.VMEM((rows_per,), jnp.int32),
            pltpu.VMEM((rows_per, D_MODEL), jnp.float32),
            pltpu.SemaphoreType.DMA,
        ],
    )
    return f(dest, eo)


def kernel(x, Wr, W1, b1, W2, b2):
    B, T, D = x.shape
    x2d = x.reshape(B * T, D)

    dest3, kc, psum, z2 = _router(x2d, Wr)
    dest = dest3.reshape(N_TOKENS)

    xb = _dispatch(dest, x2d)
    eo = _experts(xb, W1, b1, W2, b2)
    out2d = _combine(dest, eo)

    p_i = psum[0, 0, :] / jnp.float32(N_TOKENS)
    z_loss = z2[0, 0, 0] / jnp.float32(N_TOKENS)
    kcf = kc[0, 0, :]
    total_sel = jnp.maximum(jnp.sum(kcf), 1.0)
    f_i = kcf / total_sel
    aux_loss = N_EXPERTS * jnp.sum(f_i * p_i)
    total_aux = 0.01 * aux_loss + 0.001 * z_loss

    out = out2d.reshape(B, T, D)
    return out, aux_loss, z_loss, total_aux


# speedup vs baseline: 1.3000x; 1.0096x over previous
import functools

import jax
import jax.numpy as jnp
from jax import lax
from jax.experimental import pallas as pl
from jax.experimental.pallas import tpu as pltpu
from jax.experimental.pallas import tpu_sc as plsc

D_MODEL = 768
D_EXPERT = 1536
N_EXPERTS = 64
N_TOKENS = 2048
CAP = 40
N_SLOTS = N_EXPERTS * CAP
EO_ROWS = N_SLOTS + CAP
TB = 512
N_TB = N_TOKENS // TB


def _router_body(x_ref, wr_ref, dest_ref, kc_ref, psum_ref, z2_ref, cnt_ref):
    i = pl.program_id(0)
    xb = x_ref[...]
    wr = wr_ref[...]
    logits = lax.dot_general(xb, wr, (((1,), (1,)), ((), ())),
                             preferred_element_type=jnp.float32)
    mx = jnp.max(logits, axis=1, keepdims=True)
    ex = jnp.exp(logits - mx)
    s = jnp.sum(ex, axis=1, keepdims=True)
    probs = ex / s
    logz = mx + jnp.log(s)
    z2 = jnp.sum(logz * logz)

    ids = lax.broadcasted_iota(jnp.int32, (TB, N_EXPERTS), 1)
    cand = jnp.where(logits == mx, ids, N_EXPERTS)
    top1 = jnp.min(cand, axis=1, keepdims=True)
    mask = (ids == top1).astype(jnp.float32)

    r_io = lax.broadcasted_iota(jnp.int32, (TB, TB), 0)
    c_io = lax.broadcasted_iota(jnp.int32, (TB, TB), 1)
    ltri = (c_io <= r_io).astype(jnp.float32)
    incl = lax.dot_general(ltri, mask, (((1,), (0,)), ((), ())),
                           preferred_element_type=jnp.float32)

    @pl.when(i == 0)
    def _():
        cnt_ref[...] = jnp.zeros((1, N_EXPERTS), jnp.float32)

    pos = cnt_ref[...] + incl - 1.0
    cnt_ref[...] += jnp.sum(mask, axis=0, keepdims=True)

    pos_tok = jnp.sum(mask * pos, axis=1)
    kept = pos_tok < CAP
    dest = jnp.where(kept,
                     top1[:, 0] * CAP + pos_tok.astype(jnp.int32),
                     N_SLOTS)
    dest_ref[...] = dest.reshape(1, 1, TB)

    psum = jnp.sum(probs, axis=0, keepdims=True).reshape(1, 1, N_EXPERTS)
    z2b = jnp.full((1, 1, 8), z2, dtype=jnp.float32)

    @pl.when(i == 0)
    def _():
        psum_ref[...] = psum
        z2_ref[...] = z2b

    @pl.when(i > 0)
    def _():
        psum_ref[...] += psum
        z2_ref[...] += z2b

    @pl.when(i == N_TB - 1)
    def _():
        kc_ref[...] = jnp.minimum(cnt_ref[...], float(CAP)).reshape(
            1, 1, N_EXPERTS)


def _router(x2d, wr):
    return pl.pallas_call(
        _router_body,
        grid=(N_TB,),
        in_specs=[
            pl.BlockSpec((TB, D_MODEL), lambda i: (i, 0)),
            pl.BlockSpec((N_EXPERTS, D_MODEL), lambda i: (0, 0)),
        ],
        out_specs=[
            pl.BlockSpec((1, 1, TB), lambda i: (i, 0, 0)),
            pl.BlockSpec((1, 1, N_EXPERTS), lambda i: (0, 0, 0)),
            pl.BlockSpec((1, 1, N_EXPERTS), lambda i: (0, 0, 0)),
            pl.BlockSpec((1, 1, 8), lambda i: (0, 0, 0)),
        ],
        out_shape=[
            jax.ShapeDtypeStruct((N_TB, 1, TB), jnp.int32),
            jax.ShapeDtypeStruct((1, 1, N_EXPERTS), jnp.float32),
            jax.ShapeDtypeStruct((1, 1, N_EXPERTS), jnp.float32),
            jax.ShapeDtypeStruct((1, 1, 8), jnp.float32),
        ],
        scratch_shapes=[pltpu.VMEM((1, N_EXPERTS), jnp.float32)],
    )(x2d, wr)


def _dispatch_body(rows_per, dest_hbm, x_hbm, xb_hbm, idx_v, rows_v, sem):
    wid = lax.axis_index("s") * 2 + lax.axis_index("c")
    base = wid * rows_per
    pltpu.sync_copy(dest_hbm.at[pl.ds(base, rows_per)], idx_v)
    pltpu.sync_copy(x_hbm.at[pl.ds(base, rows_per)], rows_v)
    pltpu.async_copy(rows_v, xb_hbm.at[idx_v], sem).wait()


def _dispatch(dest, x2d):
    info = plsc.get_sparse_core_info()
    nw = info.num_cores * info.num_subcores
    rows_per = N_TOKENS // nw
    mesh = plsc.VectorSubcoreMesh(core_axis_name="c", subcore_axis_name="s")
    f = pl.kernel(
        functools.partial(_dispatch_body, rows_per),
        out_type=jax.ShapeDtypeStruct((EO_ROWS, D_MODEL), jnp.float32),
        mesh=mesh,
        scratch_types=[
            pltpu.VMEM((rows_per,), jnp.int32),
            pltpu.VMEM((rows_per, D_MODEL), jnp.float32),
            pltpu.SemaphoreType.DMA,
        ],
    )
    return f(dest, x2d)


def _experts_body(xb_ref, w1_ref, b1_ref, w2_ref, b2_ref, eo_ref):
    e = pl.program_id(0)
    xb = xb_ref[...]
    w1 = w1_ref[0]
    h = lax.dot_general(xb, w1, (((1,), (1,)), ((), ())),
                        preferred_element_type=jnp.float32)
    h = jnp.maximum(h + b1_ref[0], 0.0)
    w2 = w2_ref[0]
    o = lax.dot_general(h, w2, (((1,), (1,)), ((), ())),
                        preferred_element_type=jnp.float32)
    o = o + b2_ref[0]
    eo_ref[...] = jnp.where(e < N_EXPERTS, o, 0.0)


def _experts(xb, w1, b1, w2, b2):
    b1r = b1.reshape(N_EXPERTS, 1, D_EXPERT)
    b2r = b2.reshape(N_EXPERTS, 1, D_MODEL)
    return pl.pallas_call(
        _experts_body,
        grid=(N_EXPERTS + 1,),
        in_specs=[
            pl.BlockSpec((CAP, D_MODEL),
                         lambda e: (jnp.minimum(e, N_EXPERTS - 1), 0)),
            pl.BlockSpec((1, D_EXPERT, D_MODEL),
                         lambda e: (jnp.minimum(e, N_EXPERTS - 1), 0, 0)),
            pl.BlockSpec((1, 1, D_EXPERT),
                         lambda e: (jnp.minimum(e, N_EXPERTS - 1), 0, 0)),
            pl.BlockSpec((1, D_MODEL, D_EXPERT),
                         lambda e: (jnp.minimum(e, N_EXPERTS - 1), 0, 0)),
            pl.BlockSpec((1, 1, D_MODEL),
                         lambda e: (jnp.minimum(e, N_EXPERTS - 1), 0, 0)),
        ],
        out_specs=pl.BlockSpec((CAP, D_MODEL), lambda e: (e, 0)),
        out_shape=jax.ShapeDtypeStruct((EO_ROWS, D_MODEL), jnp.float32),
        compiler_params=pltpu.CompilerParams(
            vmem_limit_bytes=100 * 1024 * 1024,
        ),
    )(xb, w1, b1r, w2, b2r)


def _combine_body(rows_per, dest_hbm, eo_hbm, out_hbm, idx_v, rows_v, sem):
    wid = lax.axis_index("s") * 2 + lax.axis_index("c")
    base = wid * rows_per
    pltpu.sync_copy(dest_hbm.at[pl.ds(base, rows_per)], idx_v)
    pltpu.async_copy(eo_hbm.at[idx_v], rows_v, sem).wait()
    pltpu.sync_copy(rows_v, out_hbm.at[pl.ds(base, rows_per)])


def _combine(dest, eo):
    info = plsc.get_sparse_core_info()
    nw = info.num_cores * info.num_subcores
    rows_per = N_TOKENS // nw
    mesh = plsc.VectorSubcoreMesh(core_axis_name="c", subcore_axis_name="s")
    f = pl.kernel(
        functools.partial(_combine_body, rows_per),
        out_type=jax.ShapeDtypeStruct((N_TOKENS, D_MODEL), jnp.float32),
        mesh=mesh,
        scratch_types=[
            pltpu.VMEM((rows_per,), jnp.int32),
            pltpu.VMEM((rows_per, D_MODEL), jnp.float32),
            pltpu.SemaphoreType.DMA,
        ],
    )
    return f(dest, eo)


def kernel(x, Wr, W1, b1, W2, b2):
    B, T, D = x.shape
    x2d = x.reshape(B * T, D)

    dest3, kc, psum, z2 = _router(x2d, Wr)
    dest = dest3.reshape(N_TOKENS)

    xb = _dispatch(dest, x2d)
    eo = _experts(xb, W1, b1, W2, b2)
    out2d = _combine(dest, eo)

    p_i = psum[0, 0, :] / jnp.float32(N_TOKENS)
    z_loss = z2[0, 0, 0] / jnp.float32(N_TOKENS)
    kcf = kc[0, 0, :]
    total_sel = jnp.maximum(jnp.sum(kcf), 1.0)
    f_i = kcf / total_sel
    aux_loss = N_EXPERTS * jnp.sum(f_i * p_i)
    total_aux = 0.01 * aux_loss + 0.001 * z_loss

    out = out2d.reshape(B, T, D)
    return out, aux_loss, z_loss, total_aux


# speedup vs baseline: 1.3126x; 1.0097x over previous
import functools

import jax
import jax.numpy as jnp
from jax import lax
from jax.experimental import pallas as pl
from jax.experimental.pallas import tpu as pltpu
from jax.experimental.pallas import tpu_sc as plsc

D_MODEL = 768
D_EXPERT = 1536
N_EXPERTS = 64
N_TOKENS = 2048
CAP = 40
N_SLOTS = N_EXPERTS * CAP
EO_ROWS = N_SLOTS + CAP
TB = 512
N_TB = N_TOKENS // TB


def _router_body(x_ref, wr_ref, dest_ref, kc_ref, psum_ref, z2_ref, cnt_ref):
    i = pl.program_id(0)
    xb = x_ref[...]
    wr = wr_ref[...]
    logits = lax.dot_general(xb, wr, (((1,), (1,)), ((), ())),
                             preferred_element_type=jnp.float32)
    mx = jnp.max(logits, axis=1, keepdims=True)
    ex = jnp.exp(logits - mx)
    s = jnp.sum(ex, axis=1, keepdims=True)
    probs = ex / s
    logz = mx + jnp.log(s)
    z2 = jnp.sum(logz * logz)

    ids = lax.broadcasted_iota(jnp.int32, (TB, N_EXPERTS), 1)
    cand = jnp.where(logits == mx, ids, N_EXPERTS)
    top1 = jnp.min(cand, axis=1, keepdims=True)
    mask = (ids == top1).astype(jnp.float32)

    r_io = lax.broadcasted_iota(jnp.int32, (TB, TB), 0)
    c_io = lax.broadcasted_iota(jnp.int32, (TB, TB), 1)
    ltri = (c_io <= r_io).astype(jnp.float32)
    incl = lax.dot_general(ltri, mask, (((1,), (0,)), ((), ())),
                           preferred_element_type=jnp.float32)

    @pl.when(i == 0)
    def _():
        cnt_ref[...] = jnp.zeros((1, N_EXPERTS), jnp.float32)

    pos = cnt_ref[...] + incl - 1.0
    cnt_ref[...] += jnp.sum(mask, axis=0, keepdims=True)

    pos_tok = jnp.sum(mask * pos, axis=1)
    kept = pos_tok < CAP
    dest = jnp.where(kept,
                     top1[:, 0] * CAP + pos_tok.astype(jnp.int32),
                     N_SLOTS)
    dest_ref[...] = dest.reshape(1, 1, TB)

    psum = jnp.sum(probs, axis=0, keepdims=True).reshape(1, 1, N_EXPERTS)
    z2b = jnp.full((1, 1, 8), z2, dtype=jnp.float32)

    @pl.when(i == 0)
    def _():
        psum_ref[...] = psum
        z2_ref[...] = z2b

    @pl.when(i > 0)
    def _():
        psum_ref[...] += psum
        z2_ref[...] += z2b

    @pl.when(i == N_TB - 1)
    def _():
        kc_ref[...] = jnp.minimum(cnt_ref[...], float(CAP)).reshape(
            1, 1, N_EXPERTS)


def _router(x2d, wr):
    return pl.pallas_call(
        _router_body,
        grid=(N_TB,),
        in_specs=[
            pl.BlockSpec((TB, D_MODEL), lambda i: (i, 0)),
            pl.BlockSpec((N_EXPERTS, D_MODEL), lambda i: (0, 0)),
        ],
        out_specs=[
            pl.BlockSpec((1, 1, TB), lambda i: (i, 0, 0)),
            pl.BlockSpec((1, 1, N_EXPERTS), lambda i: (0, 0, 0)),
            pl.BlockSpec((1, 1, N_EXPERTS), lambda i: (0, 0, 0)),
            pl.BlockSpec((1, 1, 8), lambda i: (0, 0, 0)),
        ],
        out_shape=[
            jax.ShapeDtypeStruct((N_TB, 1, TB), jnp.int32),
            jax.ShapeDtypeStruct((1, 1, N_EXPERTS), jnp.float32),
            jax.ShapeDtypeStruct((1, 1, N_EXPERTS), jnp.float32),
            jax.ShapeDtypeStruct((1, 1, 8), jnp.float32),
        ],
        scratch_shapes=[pltpu.VMEM((1, N_EXPERTS), jnp.float32)],
    )(x2d, wr)


def _dispatch_body(rows_per, dest_hbm, x_hbm, xb_hbm, idx_v, rows_v, sem):
    wid = lax.axis_index("s") * 2 + lax.axis_index("c")
    base = wid * rows_per
    pltpu.sync_copy(dest_hbm.at[pl.ds(base, rows_per)], idx_v)
    pltpu.sync_copy(x_hbm.at[pl.ds(base, rows_per)], rows_v)
    pltpu.async_copy(rows_v, xb_hbm.at[idx_v], sem).wait()


def _dispatch(dest, x2d):
    info = plsc.get_sparse_core_info()
    nw = info.num_cores * info.num_subcores
    rows_per = N_TOKENS // nw
    mesh = plsc.VectorSubcoreMesh(core_axis_name="c", subcore_axis_name="s")
    f = pl.kernel(
        functools.partial(_dispatch_body, rows_per),
        out_type=jax.ShapeDtypeStruct((EO_ROWS, D_MODEL), jnp.float32),
        mesh=mesh,
        scratch_types=[
            pltpu.VMEM((rows_per,), jnp.int32),
            pltpu.VMEM((rows_per, D_MODEL), jnp.float32),
            pltpu.SemaphoreType.DMA,
        ],
    )
    return f(dest, x2d)


def _experts_body(xb_ref, w1a_ref, w1b_ref, b1_ref, w2a_ref, w2b_ref,
                  b2_ref, eo_ref):
    e = pl.program_id(0)
    xb = xb_ref[...]
    h1 = lax.dot_general(xb, w1a_ref[0], (((1,), (1,)), ((), ())),
                         preferred_element_type=jnp.float32)
    h2 = lax.dot_general(xb, w1b_ref[0], (((1,), (1,)), ((), ())),
                         preferred_element_type=jnp.float32)
    h = jnp.concatenate([h1, h2], axis=1)
    h = jnp.maximum(h + b1_ref[0], 0.0)
    o1 = lax.dot_general(h, w2a_ref[0], (((1,), (1,)), ((), ())),
                         preferred_element_type=jnp.float32)
    o2 = lax.dot_general(h, w2b_ref[0], (((1,), (1,)), ((), ())),
                         preferred_element_type=jnp.float32)
    o = jnp.concatenate([o1, o2], axis=1) + b2_ref[0]
    eo_ref[...] = jnp.where(e < N_EXPERTS, o, 0.0)


def _experts(xb, w1, b1, w2, b2):
    b1r = b1.reshape(N_EXPERTS, 1, D_EXPERT)
    b2r = b2.reshape(N_EXPERTS, 1, D_MODEL)
    H2 = D_EXPERT // 2
    M2 = D_MODEL // 2
    wspec1 = lambda j: pl.BlockSpec(
        (1, H2, D_MODEL), lambda e: (jnp.minimum(e, N_EXPERTS - 1), j, 0))
    wspec2 = lambda j: pl.BlockSpec(
        (1, M2, D_EXPERT), lambda e: (jnp.minimum(e, N_EXPERTS - 1), j, 0))
    return pl.pallas_call(
        _experts_body,
        grid=(N_EXPERTS + 1,),
        in_specs=[
            pl.BlockSpec((CAP, D_MODEL),
                         lambda e: (jnp.minimum(e, N_EXPERTS - 1), 0)),
            wspec1(0),
            wspec1(1),
            pl.BlockSpec((1, 1, D_EXPERT),
                         lambda e: (jnp.minimum(e, N_EXPERTS - 1), 0, 0)),
            wspec2(0),
            wspec2(1),
            pl.BlockSpec((1, 1, D_MODEL),
                         lambda e: (jnp.minimum(e, N_EXPERTS - 1), 0, 0)),
        ],
        out_specs=pl.BlockSpec((CAP, D_MODEL), lambda e: (e, 0)),
        out_shape=jax.ShapeDtypeStruct((EO_ROWS, D_MODEL), jnp.float32),
        compiler_params=pltpu.CompilerParams(
            vmem_limit_bytes=100 * 1024 * 1024,
        ),
    )(xb, w1, w1, b1r, w2, w2, b2r)


def _combine_body(rows_per, dest_hbm, eo_hbm, out_hbm, idx_v, rows_v, sem):
    wid = lax.axis_index("s") * 2 + lax.axis_index("c")
    base = wid * rows_per
    pltpu.sync_copy(dest_hbm.at[pl.ds(base, rows_per)], idx_v)
    pltpu.async_copy(eo_hbm.at[idx_v], rows_v, sem).wait()
    pltpu.sync_copy(rows_v, out_hbm.at[pl.ds(base, rows_per)])


def _combine(dest, eo):
    info = plsc.get_sparse_core_info()
    nw = info.num_cores * info.num_subcores
    rows_per = N_TOKENS // nw
    mesh = plsc.VectorSubcoreMesh(core_axis_name="c", subcore_axis_name="s")
    f = pl.kernel(
        functools.partial(_combine_body, rows_per),
        out_type=jax.ShapeDtypeStruct((N_TOKENS, D_MODEL), jnp.float32),
        mesh=mesh,
        scratch_types=[
            pltpu.VMEM((rows_per,), jnp.int32),
            pltpu.VMEM((rows_per, D_MODEL), jnp.float32),
            pltpu.SemaphoreType.DMA,
        ],
    )
    return f(dest, eo)


def kernel(x, Wr, W1, b1, W2, b2):
    B, T, D = x.shape
    x2d = x.reshape(B * T, D)

    dest3, kc, psum, z2 = _router(x2d, Wr)
    dest = dest3.reshape(N_TOKENS)

    xb = _dispatch(dest, x2d)
    eo = _experts(xb, W1, b1, W2, b2)
    out2d = _combine(dest, eo)

    p_i = psum[0, 0, :] / jnp.float32(N_TOKENS)
    z_loss = z2[0, 0, 0] / jnp.float32(N_TOKENS)
    kcf = kc[0, 0, :]
    total_sel = jnp.maximum(jnp.sum(kcf), 1.0)
    f_i = kcf / total_sel
    aux_loss = N_EXPERTS * jnp.sum(f_i * p_i)
    total_aux = 0.01 * aux_loss + 0.001 * z_loss

    out = out2d.reshape(B, T, D)
    return out, aux_loss, z_loss, total_aux
